# Initial kernel scaffold; baseline (speedup 1.0000x reference)
import functools

import jax
import jax.numpy as jnp
from jax import lax
from jax.experimental import pallas as pl
from jax.experimental.pallas import tpu as pltpu
from jax.experimental.pallas import tpu_sc as plsc

BINS = 100000
EMB = 16
NF = 26

NC = 2
NS = 16
LANES = 16
NW = NC * NS

CHUNK = 128


def _sc_gather_call(tot):
    bpw = tot // NW
    nchunk = bpw // CHUNK
    nvec = bpw // LANES

    mesh = plsc.VectorSubcoreMesh(core_axis_name="c", subcore_axis_name="s")

    @functools.partial(
        pl.kernel,
        out_type=jax.ShapeDtypeStruct((tot, EMB), jnp.float32),
        mesh=mesh,
        scratch_types=[
            pltpu.VMEM((bpw,), jnp.int32),
            pltpu.VMEM((bpw,), jnp.int32),
            pltpu.VMEM((bpw, EMB), jnp.float32),
            pltpu.SemaphoreType.DMA,
        ],
    )
    def sc_gather(idx_hbm, tbl_hbm, out_hbm, idx_v, fidx_v, rows_v, sem):
        wid = lax.axis_index("s") * NC + lax.axis_index("c")
        base = wid * bpw
        pltpu.sync_copy(idx_hbm.at[pl.ds(base, bpw)], idx_v)

        lane = lax.broadcasted_iota(jnp.int32, (LANES,), 0)

        def xform(j, carry):
            off = j * LANES + lane
            raw = idx_v[pl.ds(j * LANES, LANES)]
            binned = lax.rem(raw, BINS)
            field = lax.rem(off, NF)
            fidx_v[pl.ds(j * LANES, LANES)] = field * BINS + binned
            return carry

        lax.fori_loop(0, nvec, xform, 0)

        def fire(c, carry):
            pltpu.make_async_copy(
                tbl_hbm.at[fidx_v.at[pl.ds(c * CHUNK, CHUNK)]],
                rows_v.at[pl.ds(c * CHUNK, CHUNK)],
                sem,
            ).start()
            return carry

        lax.fori_loop(0, nchunk, fire, 0)
        pltpu.make_async_copy(out_hbm.at[pl.ds(base, bpw)], rows_v, sem).wait()
        pltpu.sync_copy(rows_v, out_hbm.at[pl.ds(base, bpw)])

    return sc_gather


def _mlp_body(dense_ref, embs_ref, w1a, w1b, b1, w2, b2, w3, b3, w4, b4, out_ref):
    f32 = jnp.float32
    h = jnp.dot(embs_ref[...], w1b[...], preferred_element_type=f32)
    h += jnp.dot(dense_ref[...], w1a[...], preferred_element_type=f32)
    h = jnp.maximum(h + b1[...], 0.0)
    h = jnp.maximum(jnp.dot(h, w2[...], preferred_element_type=f32) + b2[...], 0.0)
    h = jnp.maximum(jnp.dot(h, w3[...], preferred_element_type=f32) + b3[...], 0.0)
    o = jnp.dot(h, w4[...], preferred_element_type=f32) + b4[...]
    out_ref[...] = 1.0 / (1.0 + jnp.exp(-o))


def _mlp_call(dense, embs, w1a, w1b, b1, w2, b2, w3, b3, w4, b4, bt=512):
    bsz, nd = dense.shape
    demb = embs.shape[1]
    u1, u2, u3 = w2.shape[0], w3.shape[0], w4.shape[0]
    grid = (bsz // bt,)
    full = lambda shape: pl.BlockSpec(shape, lambda i: (0, 0))
    return pl.pallas_call(
        _mlp_body,
        grid=grid,
        in_specs=[
            pl.BlockSpec((bt, nd), lambda i: (i, 0)),
            pl.BlockSpec((bt, demb), lambda i: (i, 0)),
            full((nd, u1)),
            full((demb, u1)),
            full((1, u1)),
            full((u1, u2)),
            full((1, u2)),
            full((u2, u3)),
            full((1, u3)),
            full((u3, 1)),
            full((1, 1)),
        ],
        out_specs=pl.BlockSpec((bt, 1), lambda i: (i, 0)),
        out_shape=jax.ShapeDtypeStruct((bsz, 1), jnp.float32),
    )(dense, embs, w1a, w1b, b1, w2, b2, w3, b3, w4, b4)


def kernel(dense, sparse_idx, emb_table, W1, b1, W2, b2, W3, b3, W4, b4):
    bsz, nd = dense.shape
    nf, nbins, emb = emb_table.shape
    tot = bsz * nf

    idx_flat = sparse_idx.reshape(tot).astype(jnp.int32)
    tbl = emb_table.reshape(nf * nbins, emb)
    rows = _sc_gather_call(tot)(idx_flat, tbl)
    embs = rows.reshape(bsz, nf * emb)
    return _mlp_call(
        dense, embs,
        W1[:nd], W1[nd:], b1.reshape(1, -1),
        W2, b2.reshape(1, -1),
        W3, b3.reshape(1, -1),
        W4, b4.reshape(1, -1),
    )


# speedup vs baseline: 2.0891x; 2.0891x over previous
import functools

import jax
import jax.numpy as jnp
from jax import lax
from jax.experimental import pallas as pl
from jax.experimental.pallas import tpu as pltpu
from jax.experimental.pallas import tpu_sc as plsc

BINS = 100000
EMB = 16
NF = 26

NC = 2
NS = 16
LANES = 16
NW = NC * NS

CHUNK = 128


def _sc_gather_call(tot):
    bpw = tot // NW
    nchunk = bpw // CHUNK
    nvec = bpw // LANES

    mesh = plsc.VectorSubcoreMesh(core_axis_name="c", subcore_axis_name="s")

    @functools.partial(
        pl.kernel,
        out_type=jax.ShapeDtypeStruct((tot, EMB), jnp.float32),
        mesh=mesh,
        scratch_types=[
            pltpu.VMEM((bpw,), jnp.int32),
            pltpu.VMEM((bpw,), jnp.int32),
            pltpu.VMEM((bpw, EMB), jnp.float32),
            pltpu.SemaphoreType.DMA,
        ],
        compiler_params=pltpu.CompilerParams(use_tc_tiling_on_sc=False),
    )
    def sc_gather(idx_hbm, tbl_hbm, out_hbm, idx_v, fidx_v, rows_v, sem):
        wid = lax.axis_index("s") * NC + lax.axis_index("c")
        base = wid * bpw
        pltpu.sync_copy(idx_hbm.at[pl.ds(base, bpw)], idx_v)

        lane = lax.broadcasted_iota(jnp.int32, (LANES,), 0)

        def xform(j, carry):
            off = j * LANES + lane
            raw = idx_v[pl.ds(j * LANES, LANES)]
            binned = lax.rem(raw, BINS)
            field = lax.rem(off, NF)
            fidx_v[pl.ds(j * LANES, LANES)] = field * BINS + binned
            return carry

        lax.fori_loop(0, nvec, xform, 0)

        def fire(c, carry):
            pltpu.make_async_copy(
                tbl_hbm.at[fidx_v.at[pl.ds(c * CHUNK, CHUNK)]],
                rows_v.at[pl.ds(c * CHUNK, CHUNK)],
                sem,
            ).start()
            return carry

        lax.fori_loop(0, nchunk, fire, 0)
        pltpu.make_async_copy(out_hbm.at[pl.ds(base, bpw)], rows_v, sem).wait()
        pltpu.sync_copy(rows_v, out_hbm.at[pl.ds(base, bpw)])

    return sc_gather


def _mlp_body(dense_ref, embs_ref, w1a, w1b, b1, w2, b2, w3, b3, w4, b4, out_ref):
    f32 = jnp.float32
    h = jnp.dot(embs_ref[...], w1b[...], preferred_element_type=f32)
    h += jnp.dot(dense_ref[...], w1a[...], preferred_element_type=f32)
    h = jnp.maximum(h + b1[...], 0.0)
    h = jnp.maximum(jnp.dot(h, w2[...], preferred_element_type=f32) + b2[...], 0.0)
    h = jnp.maximum(jnp.dot(h, w3[...], preferred_element_type=f32) + b3[...], 0.0)
    o = jnp.dot(h, w4[...], preferred_element_type=f32) + b4[...]
    out_ref[...] = 1.0 / (1.0 + jnp.exp(-o))


def _mlp_call(dense, embs, w1a, w1b, b1, w2, b2, w3, b3, w4, b4, bt=512):
    bsz, nd = dense.shape
    demb = embs.shape[1]
    u1, u2, u3 = w2.shape[0], w3.shape[0], w4.shape[0]
    grid = (bsz // bt,)
    full = lambda shape: pl.BlockSpec(shape, lambda i: (0, 0))
    return pl.pallas_call(
        _mlp_body,
        grid=grid,
        in_specs=[
            pl.BlockSpec((bt, nd), lambda i: (i, 0)),
            pl.BlockSpec((bt, demb), lambda i: (i, 0)),
            full((nd, u1)),
            full((demb, u1)),
            full((1, u1)),
            full((u1, u2)),
            full((1, u2)),
            full((u2, u3)),
            full((1, u3)),
            full((u3, 1)),
            full((1, 1)),
        ],
        out_specs=pl.BlockSpec((bt, 1), lambda i: (i, 0)),
        out_shape=jax.ShapeDtypeStruct((bsz, 1), jnp.float32),
    )(dense, embs, w1a, w1b, b1, w2, b2, w3, b3, w4, b4)


def kernel(dense, sparse_idx, emb_table, W1, b1, W2, b2, W3, b3, W4, b4):
    bsz, nd = dense.shape
    nf, nbins, emb = emb_table.shape
    tot = bsz * nf

    idx_flat = sparse_idx.reshape(tot).astype(jnp.int32)
    tbl = emb_table.reshape(nf * nbins, emb)
    rows = _sc_gather_call(tot)(idx_flat, tbl)
    embs = rows.reshape(bsz, nf * emb)
    return _mlp_call(
        dense, embs,
        W1[:nd], W1[nd:], b1.reshape(1, -1),
        W2, b2.reshape(1, -1),
        W3, b3.reshape(1, -1),
        W4, b4.reshape(1, -1),
    )
